# Initial kernel scaffold; baseline (speedup 1.0000x reference)
import jax
import jax.numpy as jnp
from jax.experimental import pallas as pl


def kernel(x, embed1, Wg_s, Wg_d, bg, tn_w, tn_b, as_w, as_b, Wq, bq, Wk, bk, Wm, bm, incre):
    raise NotImplementedError("write your pallas kernel here")


# speedup vs baseline: 6.3561x; 6.3561x over previous
import functools
import math

import jax
import jax.numpy as jnp
from jax.experimental import pallas as pl
from jax.experimental.pallas import tpu as pltpu

B, N, DIM = 16, 1024, 64
HEADS, OUT_DIM = 4, 64
D = HEADS * OUT_DIM
NEIGHBORS = 32
U = 512
RN = 512


_MM = (((1,), (0,)), ((), ()))
_ABT = (((1,), (1,)), ((), ()))


def _dot(a, b, dn=_MM):
    return jax.lax.dot_general(_rtne_bf16(a), _rtne_bf16(b), dn,
                               preferred_element_type=jnp.float32)


def _rtne_bf16(a):
    u = jax.lax.bitcast_convert_type(a, jnp.uint32)
    lsb = (u >> 16) & jnp.uint32(1)
    r = (u + jnp.uint32(0x7FFF) + lsb) & jnp.uint32(0xFFFF0000)
    return jax.lax.bitcast_convert_type(r, jnp.float32)


def _fused_body(npre_ref, emb_ref, wgs_ref, wgd_ref, tnw_ref, tnb_ref,
                bg_ref, wq_ref, bq_ref, wk_ref, bk_ref, wm_ref, bm_ref,
                out_ref, kscr):
    j = pl.program_id(1)

    np_blk = npre_ref[0]
    mu = jnp.mean(np_blk, axis=-1, keepdims=True)
    var = jnp.mean((np_blk - mu) ** 2, axis=-1, keepdims=True)
    ni = (np_blk - mu) / jnp.sqrt(var + 1e-5) * tnw_ref[0] + tnb_ref[0]

    emb = emb_ref[...]
    z = jax.nn.sigmoid(
        _dot(emb, wgs_ref[...]) + _dot(ni, wgd_ref[...]) + bg_ref[0])
    nv = z * emb + (1.0 - z) * ni + emb

    q = _dot(nv, wq_ref[...]) + bq_ref[0]

    @pl.when(j == 0)
    def _():
        kscr[...] = _dot(nv, wk_ref[...]) + bk_ref[0]

    kk = kscr[...]
    inv_sqrt_d = 1.0 / math.sqrt(OUT_DIM)
    atts = []
    for h in range(HEADS):
        qh = q[:, h * OUT_DIM:(h + 1) * OUT_DIM]
        kh = kk[:, h * OUT_DIM:(h + 1) * OUT_DIM]
        atts.append(_dot(qh, kh, _ABT) * inv_sqrt_d)

    attb = [_rtne_bf16(a) for a in atts]
    s = atts[0] + atts[1] + atts[2] + atts[3]
    for o in range(HEADS):
        p0 = wm_ref[0, o * HEADS + 0] * attb[0] + wm_ref[0, o * HEADS + 1] * attb[1]
        p1 = wm_ref[0, o * HEADS + 2] * attb[2] + wm_ref[0, o * HEADS + 3] * attb[3]
        s = s + jnp.maximum(p0 + p1 + bm_ref[0, o], 0.0)

    def body(_, cur):
        mx = jnp.max(cur, axis=1, keepdims=True)
        return jnp.where(cur >= mx, -jnp.inf, cur)

    cur = jax.lax.fori_loop(0, NEIGHBORS - 1, body, s)
    t = jnp.max(cur, axis=1, keepdims=True)

    out_ref[0, :, :U] = jnp.where(s >= t, s, 0.0)
    out_ref[0, :, U:] = jnp.zeros((RN, N - U), jnp.float32)


@functools.partial(jax.jit, static_argnames=())
def kernel(x, embed1, Wg_s, Wg_d, bg, tn_w, tn_b, as_w, as_b, Wq, bq, Wk, bk,
           Wm, bm, incre):
    del as_w, as_b

    ch = jnp.arange(DIM)
    x_in = jnp.where(ch[None, None, :] < 24, x, 0.0)
    p2 = jnp.where(ch[None, None, :] == 0, x[:, :, 24:25], 0.0)
    mu2 = jnp.mean(p2, axis=-1, keepdims=True)
    var2 = jnp.mean((p2 - mu2) ** 2, axis=-1, keepdims=True)
    fusion0 = (p2 - mu2) / jnp.sqrt(var2 + 1e-5) * tn_w + tn_b
    npre = jnp.where(incre != 0, x_in + fusion0, x)

    grid = (B, N // RN)
    full = lambda shape: pl.BlockSpec(shape, lambda b, j: tuple(0 for _ in shape))
    out = pl.pallas_call(
        _fused_body,
        grid=grid,
        in_specs=[
            pl.BlockSpec((1, RN, DIM), lambda b, j: (b, j, 0)),
            pl.BlockSpec((RN, DIM), lambda b, j: (j, 0)),
            full((DIM, DIM)),
            full((DIM, DIM)),
            full((1, DIM)),
            full((1, DIM)),
            full((1, DIM)),
            full((DIM, D)),
            full((1, D)),
            full((DIM, D)),
            full((1, D)),
            pl.BlockSpec(memory_space=pltpu.SMEM),
            pl.BlockSpec(memory_space=pltpu.SMEM),
        ],
        out_specs=pl.BlockSpec((1, RN, N), lambda b, j: (b, j, 0)),
        out_shape=jax.ShapeDtypeStruct((B, N, N), jnp.float32),
        scratch_shapes=[pltpu.VMEM((RN, D), jnp.float32)],
        compiler_params=pltpu.CompilerParams(
            dimension_semantics=("arbitrary", "arbitrary")),
    )(npre, embed1, Wg_s, Wg_d,
      tn_w.reshape(1, DIM), tn_b.reshape(1, DIM), bg.reshape(1, DIM),
      Wq, bq.reshape(1, D), Wk, bk.reshape(1, D),
      Wm.astype(jnp.bfloat16).astype(jnp.float32).reshape(1, HEADS * HEADS),
      bm.reshape(1, HEADS))
    return out
